# baseline (speedup 1.0000x reference)
import functools

import jax
import jax.numpy as jnp
from jax import lax
from jax.experimental import pallas as pl
from jax.experimental.pallas import tpu as pltpu, tpu_sc as plsc


@functools.cache
def _build_gather(T, S, V, D):
    info = plsc.get_sparse_core_info()
    NC, NS = info.num_cores, info.num_subcores
    NW = NC * NS
    DB = D // 8
    CT = 512
    NTB = T // CT
    CB4 = CT // 128
    NU = S * NTB
    assert NU % NW == 0
    u_per_w = NU // NW
    ROWLEN = (T // 128) * 8 * 128
    UNIT = CB4 * 8 * 128
    mesh = plsc.VectorSubcoreMesh(core_axis_name="c", subcore_axis_name="s")

    @functools.partial(
        pl.kernel,
        mesh=mesh,
        compiler_params=pltpu.CompilerParams(
            use_tc_tiling_on_sc=False, needs_layout_passes=False
        ),
        out_type=jax.ShapeDtypeStruct((S * DB * ROWLEN,), jnp.float32),
        scratch_types=[
            pltpu.VMEM((CT,), jnp.int32),
            pltpu.VMEM((CT, D), jnp.float32),
            pltpu.VMEM((DB * UNIT,), jnp.float32),
            pltpu.SemaphoreType.DMA,
        ],
    )
    def gather_kernel(table_hbm, xt_hbm, out_hbm, idx_v, rows_v, trans_v, gsem):
        wid = lax.axis_index("s") * NC + lax.axis_index("c")
        iota = lax.iota(jnp.int32, 16)
        abase = (iota & 7) * 128 + (iota >> 3) * UNIT

        def unit(i, carry):
            uid = wid * u_per_w + i
            s = uid >> 5
            tb = uid & (NTB - 1)
            t0 = tb * CT
            pltpu.sync_copy(xt_hbm.at[s, pl.ds(t0, CT)], idx_v)
            pltpu.async_copy(table_hbm.at[idx_v], rows_v, gsem).wait()

            def trans_block(j, carry2):
                tbase = j * 8
                scal = (tbase >> 7) * 1024 + (tbase & 127)
                for k in range(8):
                    t = tbase + k
                    a = abase + (scal + k)
                    for h in range(D // 16):
                        vals = rows_v[t, pl.ds(h * 16, 16)]
                        plsc.store_scatter(trans_v, [a + (h * 2) * UNIT], vals)
                return carry2

            lax.fori_loop(0, CT // 8, trans_block, 0)
            for dblk in range(DB):
                pltpu.sync_copy(
                    trans_v.at[pl.ds(dblk * UNIT, UNIT)],
                    out_hbm.at[pl.ds((s * DB + dblk) * ROWLEN + tb * UNIT, UNIT)],
                )
            return carry

        lax.fori_loop(0, u_per_w, unit, 0)

    return gather_kernel


def kernel(x_T, weight_VxD):
    T, S = x_T.shape
    V, D = weight_VxD.shape
    DB = D // 8
    xt = x_T.T
    o1 = _build_gather(T, S, V, D)(weight_VxD, xt)
    o5 = o1.reshape(S, DB, T // 128, 8, 128)
    return o5.transpose(2, 4, 0, 1, 3).reshape(T, S, D)


# speedup vs baseline: 1.0884x; 1.0884x over previous
import functools

import jax
import jax.numpy as jnp
from jax import lax
from jax.experimental import pallas as pl
from jax.experimental.pallas import tpu as pltpu, tpu_sc as plsc


@functools.cache
def _build_gather(T, S, V, D):
    info = plsc.get_sparse_core_info()
    NC, NS = info.num_cores, info.num_subcores
    NW = NC * NS
    DB = D // 8
    CT = 512
    NTB = T // CT
    NU = S * NTB
    assert NU % NW == 0
    u_per_w = NU // NW
    assert u_per_w % 2 == 0
    ROWLEN = (T // 128) * 8 * 128
    UNIT = (CT // 128) * 8 * 128
    mesh = plsc.VectorSubcoreMesh(core_axis_name="c", subcore_axis_name="s")

    @functools.partial(
        pl.kernel,
        mesh=mesh,
        compiler_params=pltpu.CompilerParams(
            use_tc_tiling_on_sc=False, needs_layout_passes=False
        ),
        out_type=jax.ShapeDtypeStruct((S * DB * ROWLEN,), jnp.float32),
        scratch_types=[
            pltpu.VMEM((2, CT), jnp.int32),
            pltpu.VMEM((2, CT, D), jnp.float32),
            pltpu.VMEM((2, DB * UNIT), jnp.float32),
            pltpu.SemaphoreType.DMA,
            pltpu.SemaphoreType.DMA,
            pltpu.SemaphoreType.DMA,
            pltpu.SemaphoreType.DMA,
            pltpu.SemaphoreType.DMA,
            pltpu.SemaphoreType.DMA,
        ],
    )
    def gather_kernel(
        table_hbm, xt_hbm, out_hbm, idx_v, rows_v, trans_v,
        is0, is1, gs0, gs1, os0, os1,
    ):
        wid = lax.axis_index("s") * NC + lax.axis_index("c")
        u0 = wid * u_per_w
        iota = lax.iota(jnp.int32, 16)
        abase = (iota & 7) * 128 + (iota >> 3) * UNIT
        isem = (is0, is1)
        gsem = (gs0, gs1)
        osem = (os0, os1)

        def idx_copy(u, b):
            return pltpu.make_async_copy(
                xt_hbm.at[u >> 5, pl.ds((u & (NTB - 1)) * CT, CT)],
                idx_v.at[b],
                isem[b],
            )

        def gather_copy(b):
            return pltpu.make_async_copy(
                table_hbm.at[idx_v.at[b]], rows_v.at[b], gsem[b]
            )

        def out_copy(u, b, dblk):
            s = u >> 5
            tb = u & (NTB - 1)
            return pltpu.make_async_copy(
                trans_v.at[b, pl.ds(dblk * UNIT, UNIT)],
                out_hbm.at[pl.ds((s * DB + dblk) * ROWLEN + tb * UNIT, UNIT)],
                osem[b],
            )

        def transpose(b):
            def trans_block(j, carry2):
                tbase = j * 8
                scal = (tbase >> 7) * 1024 + (tbase & 127)
                for k in range(8):
                    t = tbase + k
                    a = abase + (scal + k)
                    for h in range(D // 16):
                        vals = rows_v[b, t, pl.ds(h * 16, 16)]
                        plsc.store_scatter(
                            trans_v.at[b], [a + (h * 2) * UNIT], vals
                        )
                return carry2

            lax.fori_loop(0, CT // 8, trans_block, 0)

        idx_copy(u0, 0).start()
        idx_copy(u0, 0).wait()
        gather_copy(0).start()
        idx_copy(u0 + 1, 1).start()

        def step(k, carry):
            for par in (0, 1):
                uo = k * 2 + par
                u = u0 + uo
                X, Y = par, 1 - par
                gather_copy(X).wait()

                @pl.when(uo + 1 < u_per_w)
                def _():
                    idx_copy(u + 1, Y).wait()
                    gather_copy(Y).start()

                    @pl.when(uo + 2 < u_per_w)
                    def _():
                        idx_copy(u + 2, X).start()

                @pl.when(uo >= 2)
                def _():
                    for dblk in range(DB):
                        out_copy(u - 2, X, dblk).wait()

                transpose(X)
                for dblk in range(DB):
                    out_copy(u, X, dblk).start()
            return carry

        lax.fori_loop(0, u_per_w // 2, step, 0)
        for par in (0, 1):
            u = u0 + u_per_w - 2 + par
            for dblk in range(DB):
                out_copy(u, par, dblk).wait()

    return gather_kernel


def kernel(x_T, weight_VxD):
    T, S = x_T.shape
    V, D = weight_VxD.shape
    DB = D // 8
    xt = x_T.T
    o1 = _build_gather(T, S, V, D)(weight_VxD, xt)
    o5 = o1.reshape(S, DB, T // 128, 8, 128)
    return o5.transpose(2, 4, 0, 1, 3).reshape(T, S, D)
